# Initial kernel scaffold; baseline (speedup 1.0000x reference)
import jax
import jax.numpy as jnp
from jax.experimental import pallas as pl


def kernel(node_feat, flat, edge_index, W_ih, W_hh, b_ih, b_hh, emb, Wf, bf, Wl1, bl1, Wr1, Wl2, bl2, Wr2, Wo, bo):
    raise NotImplementedError("write your pallas kernel here")


# speedup vs baseline: 4.6152x; 4.6152x over previous
import jax
import jax.numpy as jnp
from jax.experimental import pallas as pl
from jax.experimental.pallas import tpu as pltpu

N = 100
T = 64
F = 128
H = 256
EMB = 64
FLAT_IN = 32
FLAT_OUT = 64
GNN_HID = 256
GNN_OUT = 128
E = 800
NP = 128


def _fused_body(xT_ref, flat_ref, emb_ref, edge_ref,
                WihT_ref, WhhT_ref, bih_ref, bhh_ref,
                Wf_ref, bf_ref, Wl1_ref, bl1_ref, Wr1_ref,
                Wl2_ref, bl2_ref, Wr2_ref, Wo_ref, bo_ref,
                out_ref, gi_ref):
    f32 = jnp.float32
    gi_ref[...] = (
        jnp.dot(xT_ref[...], WihT_ref[...], preferred_element_type=f32)
        + bih_ref[...]
    )

    WhhT = WhhT_ref[...]
    bhh = bhh_ref[...]

    def step(t, h):
        gi = gi_ref[pl.ds(t * NP, NP), :]
        gh = jnp.dot(h, WhhT, preferred_element_type=f32) + bhh
        r = jax.nn.sigmoid(gi[:, 0:H] + gh[:, 0:H])
        z = jax.nn.sigmoid(gi[:, H:2 * H] + gh[:, H:2 * H])
        n = jnp.tanh(gi[:, 2 * H:3 * H] + r * gh[:, 2 * H:3 * H])
        return (1.0 - z) * n + z * h

    h = jax.lax.fori_loop(0, T, step, jnp.zeros((NP, H), f32))

    flat_enc = (
        jnp.dot(flat_ref[...], Wf_ref[...], preferred_element_type=f32)
        + bf_ref[...]
    )
    gnn_in = jnp.concatenate([h, flat_enc, emb_ref[...]], axis=1)

    src = edge_ref[0:1, :]
    dst = edge_ref[1:2, :]
    iota = jax.lax.broadcasted_iota(jnp.int32, (NP, E), 0)
    oh_src = (iota == src).astype(f32)
    oh_dst = (iota == dst).astype(f32)
    A = jax.lax.dot_general(oh_dst, oh_src,
                            (((1,), (1,)), ((), ())),
                            preferred_element_type=f32)
    cnt = jnp.sum(oh_dst, axis=1, keepdims=True)
    denom = jnp.maximum(cnt, 1.0)

    mean1 = jnp.dot(A, gnn_in, preferred_element_type=f32) / denom
    h1 = jax.nn.relu(
        jnp.dot(mean1, Wl1_ref[...], preferred_element_type=f32)
        + bl1_ref[...]
        + jnp.dot(gnn_in, Wr1_ref[...], preferred_element_type=f32)
    )
    mean2 = jnp.dot(A, h1, preferred_element_type=f32) / denom
    g2 = (
        jnp.dot(mean2, Wl2_ref[...], preferred_element_type=f32)
        + bl2_ref[...]
        + jnp.dot(h1, Wr2_ref[...], preferred_element_type=f32)
    )

    cat = jnp.concatenate([g2, h], axis=1)
    logits = jnp.dot(cat, Wo_ref[...], preferred_element_type=f32) + bo_ref[...]
    out_ref[...] = jax.nn.sigmoid(logits)


def kernel(node_feat, flat, edge_index, W_ih, W_hh, b_ih, b_hh, emb,
           Wf, bf, Wl1, bl1, Wr1, Wl2, bl2, Wr2, Wo, bo):
    f32 = jnp.float32
    xT = jnp.transpose(node_feat, (1, 0, 2))
    xT = jnp.pad(xT, ((0, 0), (0, NP - N), (0, 0)))
    xT = xT.reshape(T * NP, F)
    flat_p = jnp.pad(flat, ((0, NP - N), (0, 0)))
    emb_p = jnp.pad(emb, ((0, NP - N), (0, 0)))

    out = pl.pallas_call(
        _fused_body,
        out_shape=jax.ShapeDtypeStruct((NP, 1), f32),
        scratch_shapes=[pltpu.VMEM((T * NP, 3 * H), f32)],
    )(
        xT, flat_p, emb_p, edge_index,
        W_ih.T, W_hh.T, b_ih.reshape(1, -1), b_hh.reshape(1, -1),
        Wf, bf.reshape(1, -1),
        Wl1, bl1.reshape(1, -1), Wr1,
        Wl2, bl2.reshape(1, -1), Wr2,
        Wo, bo.reshape(1, 1),
    )
    return out[:N, 0]


# speedup vs baseline: 4.8362x; 1.0479x over previous
import jax
import jax.numpy as jnp
from jax.experimental import pallas as pl
from jax.experimental.pallas import tpu as pltpu

N = 100
T = 64
F = 128
H = 256
EMB = 64
FLAT_IN = 32
FLAT_OUT = 64
GNN_HID = 256
GNN_OUT = 128
E = 800
NP = 128


def _fused_body(xT_ref, flat_ref, emb_ref, edge_ref,
                WihT_ref, WhhT_ref, bih_ref, bhh_ref,
                Wf_ref, bf_ref, Wl1_ref, bl1_ref, Wr1_ref,
                Wl2_ref, bl2_ref, Wr2_ref, Wo_ref, bo_ref,
                out_ref, gi_ref):
    f32 = jnp.float32
    gi_ref[...] = (
        jnp.dot(xT_ref[...], WihT_ref[...], preferred_element_type=f32)
        + bih_ref[...]
    )

    WhhT = WhhT_ref[...]
    bhh = bhh_ref[...]

    def step(t, h):
        gi = gi_ref[pl.ds(t * NP, NP), :]
        gh = jnp.dot(h.astype(jnp.bfloat16), WhhT,
                     preferred_element_type=f32) + bhh
        r = jax.nn.sigmoid(gi[:, 0:H] + gh[:, 0:H])
        z = jax.nn.sigmoid(gi[:, H:2 * H] + gh[:, H:2 * H])
        n = jnp.tanh(gi[:, 2 * H:3 * H] + r * gh[:, 2 * H:3 * H])
        return (1.0 - z) * n + z * h

    h = jax.lax.fori_loop(0, T, step, jnp.zeros((NP, H), f32))

    flat_enc = (
        jnp.dot(flat_ref[...], Wf_ref[...], preferred_element_type=f32)
        + bf_ref[...]
    )
    gnn_in = jnp.concatenate([h, flat_enc, emb_ref[...]], axis=1)

    src = edge_ref[0:1, :]
    dst = edge_ref[1:2, :]
    iota = jax.lax.broadcasted_iota(jnp.int32, (NP, E), 0)
    oh_src = (iota == src).astype(f32)
    oh_dst = (iota == dst).astype(f32)
    A = jax.lax.dot_general(oh_dst, oh_src,
                            (((1,), (1,)), ((), ())),
                            preferred_element_type=f32)
    cnt = jnp.sum(oh_dst, axis=1, keepdims=True)
    denom = jnp.maximum(cnt, 1.0)

    mean1 = jnp.dot(A, gnn_in, preferred_element_type=f32) / denom
    h1 = jax.nn.relu(
        jnp.dot(mean1, Wl1_ref[...], preferred_element_type=f32)
        + bl1_ref[...]
        + jnp.dot(gnn_in, Wr1_ref[...], preferred_element_type=f32)
    )
    mean2 = jnp.dot(A, h1, preferred_element_type=f32) / denom
    g2 = (
        jnp.dot(mean2, Wl2_ref[...], preferred_element_type=f32)
        + bl2_ref[...]
        + jnp.dot(h1, Wr2_ref[...], preferred_element_type=f32)
    )

    cat = jnp.concatenate([g2, h], axis=1)
    logits = jnp.dot(cat, Wo_ref[...], preferred_element_type=f32) + bo_ref[...]
    out_ref[...] = jax.nn.sigmoid(logits)


def kernel(node_feat, flat, edge_index, W_ih, W_hh, b_ih, b_hh, emb,
           Wf, bf, Wl1, bl1, Wr1, Wl2, bl2, Wr2, Wo, bo):
    f32 = jnp.float32
    xT = jnp.transpose(node_feat, (1, 0, 2))
    xT = jnp.pad(xT, ((0, 0), (0, NP - N), (0, 0)))
    xT = xT.reshape(T * NP, F)
    flat_p = jnp.pad(flat, ((0, NP - N), (0, 0)))
    emb_p = jnp.pad(emb, ((0, NP - N), (0, 0)))

    out = pl.pallas_call(
        _fused_body,
        out_shape=jax.ShapeDtypeStruct((NP, 1), f32),
        scratch_shapes=[pltpu.VMEM((T * NP, 3 * H), f32)],
    )(
        xT.astype(jnp.bfloat16), flat_p, emb_p, edge_index,
        W_ih.T.astype(jnp.bfloat16), W_hh.T.astype(jnp.bfloat16),
        b_ih.reshape(1, -1), b_hh.reshape(1, -1),
        Wf, bf.reshape(1, -1),
        Wl1, bl1.reshape(1, -1), Wr1,
        Wl2, bl2.reshape(1, -1), Wr2,
        Wo, bo.reshape(1, 1),
    )
    return out[:N, 0]
